# Initial kernel scaffold; baseline (speedup 1.0000x reference)
import jax
import jax.numpy as jnp
from jax.experimental import pallas as pl


def kernel(all_rois, gt_boxes, num_boxes, gt_poses):
    raise NotImplementedError("write your pallas kernel here")


# speedup vs baseline: 1.6240x; 1.6240x over previous
import functools

import jax
import jax.numpy as jnp
from jax import lax
from jax.experimental import pallas as pl
from jax.experimental.pallas import tpu as pltpu
from jax.experimental.pallas import tpu_sc as plsc

N_ROWS = 2048
N_GT = 20
ROIS_PER_IMAGE = 128
FG_ROIS_PER_IMAGE = 32
NSC = 16
ROWS_PER_SC = N_ROWS // NSC
VPT = ROWS_PER_SC // 16


def _iou_body(x1_ref, y1_ref, x2_ref, y2_ref, gt_ref, mo_ref, bj_ref):
    rx1 = x1_ref[...]
    ry1 = y1_ref[...]
    rx2 = x2_ref[...]
    ry2 = y2_ref[...]
    ra = (rx2 - rx1 + 1.0) * (ry2 - ry1 + 1.0)
    mo = jnp.full(rx1.shape, -1.0, jnp.float32)
    bj = jnp.zeros(rx1.shape, jnp.int32)
    for j in range(N_GT):
        gx1 = gt_ref[0, j]
        gy1 = gt_ref[1, j]
        gx2 = gt_ref[2, j]
        gy2 = gt_ref[3, j]
        ga = (gx2 - gx1 + 1.0) * (gy2 - gy1 + 1.0)
        iw = jnp.maximum(jnp.minimum(rx2, gx2) - jnp.maximum(rx1, gx1) + 1.0, 0.0)
        ih = jnp.maximum(jnp.minimum(ry2, gy2) - jnp.maximum(ry1, gy1) + 1.0, 0.0)
        inter = iw * ih
        union = ra + ga - inter
        ov = inter / jnp.maximum(union, 1e-8)
        upd = ov > mo
        mo = jnp.where(upd, ov, mo)
        bj = jnp.where(upd, j, bj)
    mo_ref[...] = mo
    bj_ref[...] = bj


def _run_iou(bx1, by1, bx2, by2, gt_pack):
    shp = (NSC, ROWS_PER_SC)
    return pl.pallas_call(
        _iou_body,
        in_specs=[
            pl.BlockSpec(memory_space=pltpu.VMEM),
            pl.BlockSpec(memory_space=pltpu.VMEM),
            pl.BlockSpec(memory_space=pltpu.VMEM),
            pl.BlockSpec(memory_space=pltpu.VMEM),
            pl.BlockSpec(memory_space=pltpu.SMEM),
        ],
        out_specs=[
            pl.BlockSpec(memory_space=pltpu.VMEM),
            pl.BlockSpec(memory_space=pltpu.VMEM),
        ],
        out_shape=[
            jax.ShapeDtypeStruct(shp, jnp.float32),
            jax.ShapeDtypeStruct(shp, jnp.int32),
        ],
    )(bx1.reshape(shp), by1.reshape(shp), bx2.reshape(shp), by2.reshape(shp),
      gt_pack)


def _excl_rank(x, scratch, base, iota):
    inc = x
    for step, kk in enumerate((1, 2, 4, 8)):
        off = base + step * 16
        scratch[pl.ds(off, 16)] = inc
        sh = plsc.load_gather(scratch, [off + jnp.maximum(iota - kk, 0)])
        inc = inc + jnp.where(iota >= kk, sh, 0)
    return inc - x


def _sc_body(mo_hbm, bj_hbm, bx1_hbm, by1_hbm, bx2_hbm, by2_hbm, gt_hbm,
             out_hbm,
             mo_v, loc_both, tmp_v,
             stg_v, bj_v,
             bx1_v, by1_v, bx2_v, by2_v, gt_v,
             cum_v, out_v,
             stg_s):
    cid = lax.axis_index("c")
    sid = lax.axis_index("s")
    iota = lax.iota(jnp.int32, 16)

    @pl.when(cid == 0)
    def _phase1():
        pltpu.sync_copy(mo_hbm.at[sid], mo_v)
        neg1 = jnp.full((16,), -1, jnp.int32)
        for r in range(2 * VPT):
            loc_both[pl.ds(r * 16, 16)] = neg1
        fg_c = jnp.int32(0)
        bg_c = jnp.int32(0)
        for r in range(VPT):
            m = mo_v[pl.ds(r * 16, 16)]
            gidx = sid * ROWS_PER_SC + r * 16 + iota
            fgm = m >= 0.5
            bgm = jnp.logical_and(m < 0.5, m >= 0.1)
            frank = jnp.clip(fg_c + _excl_rank(fgm.astype(jnp.int32),
                                               tmp_v, 0, iota),
                             0, ROWS_PER_SC - 1)
            brank = jnp.clip(bg_c + _excl_rank(bgm.astype(jnp.int32),
                                               tmp_v, 64, iota),
                             0, ROWS_PER_SC - 1)
            plsc.store_scatter(loc_both, [frank], gidx, mask=fgm)
            plsc.store_scatter(loc_both, [ROWS_PER_SC + brank], gidx,
                               mask=bgm)
            fg_c = fg_c + plsc.all_reduce_population_count(fgm)[0]
            bg_c = bg_c + plsc.all_reduce_population_count(bgm)[0]
        pltpu.sync_copy(loc_both, stg_s.at[sid])

    plsc.subcore_barrier()

    @pl.when(jnp.logical_and(cid == 0, sid == 0))
    def _phase2():
        pltpu.sync_copy(stg_s, stg_v)
        pltpu.sync_copy(bj_hbm, bj_v)
        pltpu.sync_copy(bx1_hbm, bx1_v)
        pltpu.sync_copy(by1_hbm, by1_v)
        pltpu.sync_copy(bx2_hbm, bx2_v)
        pltpu.sync_copy(by2_hbm, by2_v)
        pltpu.sync_copy(gt_hbm, gt_v)
        cf = []
        cb = []
        for s in range(NSC):
            f_s = jnp.int32(0)
            b_s = jnp.int32(0)
            for r in range(VPT):
                vf = stg_v[s, pl.ds(r * 16, 16)]
                vb = stg_v[s, pl.ds(ROWS_PER_SC + r * 16, 16)]
                f_s = f_s + plsc.all_reduce_population_count(vf >= 0)[0]
                b_s = b_s + plsc.all_reduce_population_count(vb >= 0)[0]
            cf.append(f_s)
            cb.append(b_s)
        cumf = []
        cumb = []
        fgo = jnp.int32(0)
        bgo = jnp.int32(0)
        cumf_vec = jnp.zeros((16,), jnp.int32)
        cumb_vec = jnp.zeros((16,), jnp.int32)
        for s in range(NSC):
            cumf.append(fgo)
            cumb.append(bgo)
            cumf_vec = jnp.where(iota == s, fgo, cumf_vec)
            cumb_vec = jnp.where(iota == s, bgo, cumb_vec)
            fgo = fgo + cf[s]
            bgo = bgo + cb[s]
        fg_total = fgo
        bg_total = bgo
        cum_v[pl.ds(0, 16)] = cumf_vec
        cum_v[pl.ds(16, 16)] = cumb_vec

        def _route_fg(j):
            cnum = jnp.zeros((16,), jnp.int32)
            for s in range(1, NSC):
                cnum = cnum + (j >= cumf[s]).astype(jnp.int32)
            pos = j - plsc.load_gather(cum_v, [cnum])
            return plsc.load_gather(
                stg_v, [cnum, jnp.clip(pos, 0, ROWS_PER_SC - 1)])

        def _route_bg(j):
            cnum = jnp.zeros((16,), jnp.int32)
            for s in range(1, NSC):
                cnum = cnum + (j >= cumb[s]).astype(jnp.int32)
            pos = j - plsc.load_gather(cum_v, [16 + cnum])
            return plsc.load_gather(
                stg_v, [cnum, jnp.clip(ROWS_PER_SC + pos,
                                       ROWS_PER_SC, 2 * ROWS_PER_SC - 1)])

        has_fg = fg_total > 0
        has_bg = bg_total > 0
        both = jnp.logical_and(has_fg, has_bg)
        safe_fg = jnp.maximum(fg_total, 1)
        safe_bg = jnp.maximum(bg_total, 1)
        fg_n1 = jnp.minimum(jnp.int32(FG_ROIS_PER_IMAGE), fg_total)
        fg_n = jnp.where(both, fg_n1,
                         jnp.where(has_fg, jnp.int32(ROIS_PER_IMAGE),
                                   jnp.int32(0)))

        rt0 = lax.axis_index("c")
        for k in range(ROIS_PER_IMAGE // 16):
            i = (rt0 + k * 16) + iota
            use_fg = i < fg_n1
            fg_idx = jnp.where(both, jnp.where(use_fg, i, 0),
                               lax.rem(i, safe_fg))
            bgq = jnp.where(use_fg, 0, i - fg_n1)
            bg_idx = jnp.where(both, lax.rem(bgq, safe_bg),
                               lax.rem(i, safe_bg))
            fg_val = _route_fg(jnp.clip(fg_idx, 0, ROIS_PER_IMAGE - 1))
            bg_val = _route_bg(jnp.clip(bg_idx, 0, ROIS_PER_IMAGE - 1))
            keep = jnp.where(both, jnp.where(use_fg, fg_val, bg_val),
                             jnp.where(has_fg, fg_val, bg_val))
            keep = jnp.clip(keep, 0, N_ROWS - 1)
            kx1 = plsc.load_gather(bx1_v, [keep])
            ky1 = plsc.load_gather(by1_v, [keep])
            kx2 = plsc.load_gather(bx2_v, [keep])
            ky2 = plsc.load_gather(by2_v, [keep])
            a = plsc.load_gather(
                bj_v, [lax.shift_right_logical(keep, 7),
                       lax.bitwise_and(keep, 127)])
            a = jnp.clip(a, 0, 31)
            lab = plsc.load_gather(gt_v, [jnp.full((16,), 4, jnp.int32), a])
            pose = plsc.load_gather(gt_v, [jnp.full((16,), 5, jnp.int32), a])
            ggx1 = plsc.load_gather(gt_v, [jnp.full((16,), 0, jnp.int32), a])
            ggy1 = plsc.load_gather(gt_v, [jnp.full((16,), 1, jnp.int32), a])
            ggx2 = plsc.load_gather(gt_v, [jnp.full((16,), 2, jnp.int32), a])
            ggy2 = plsc.load_gather(gt_v, [jnp.full((16,), 3, jnp.int32), a])
            labz = jnp.where(i < fg_n, lab, 0.0)
            sl = pl.ds(k * 16, 16)
            out_v[0, sl] = kx1
            out_v[1, sl] = ky1
            out_v[2, sl] = kx2
            out_v[3, sl] = ky2
            out_v[4, sl] = labz
            out_v[5, sl] = pose
            out_v[6, sl] = ggx1
            out_v[7, sl] = ggy1
            out_v[8, sl] = ggx2
            out_v[9, sl] = ggy2
        pltpu.sync_copy(out_v, out_hbm)


def _run_sc(mo, bj, bx1, by1, bx2, by2, gt_pack):
    mesh = plsc.VectorSubcoreMesh(core_axis_name="c", subcore_axis_name="s",
                                  num_cores=2, num_subcores=16)
    f32 = jnp.float32
    i32 = jnp.int32
    kern = pl.kernel(
        _sc_body,
        out_type=jax.ShapeDtypeStruct((10, ROIS_PER_IMAGE), f32),
        mesh=mesh,
        compiler_params=pltpu.CompilerParams(needs_layout_passes=False),
        scratch_types=[
            pltpu.VMEM((ROWS_PER_SC,), f32),
            pltpu.VMEM((2 * ROWS_PER_SC,), i32),
            pltpu.VMEM((128,), i32),
            pltpu.VMEM((NSC, 2 * ROWS_PER_SC), i32),
            pltpu.VMEM((NSC, ROWS_PER_SC), i32),
            pltpu.VMEM((N_ROWS,), f32),
            pltpu.VMEM((N_ROWS,), f32),
            pltpu.VMEM((N_ROWS,), f32),
            pltpu.VMEM((N_ROWS,), f32),
            pltpu.VMEM((8, 32), f32),
            pltpu.VMEM((32,), i32),
            pltpu.VMEM((10, ROIS_PER_IMAGE), f32),
            pltpu.VMEM_SHARED((NSC, 2 * ROWS_PER_SC), i32),
        ],
    )
    return kern(mo, bj, bx1, by1, bx2, by2, gt_pack)


def _transform_body(x_ref, o_ref):
    x = x_ref[...]
    ex1, ey1, ex2, ey2 = x[0:1], x[1:2], x[2:3], x[3:4]
    lab = x[4:5]
    gx1, gy1, gx2, gy2 = x[6:7], x[7:8], x[8:9], x[9:10]
    ew = ex2 - ex1 + 1.0
    eh = ey2 - ey1 + 1.0
    ecx = ex1 + 0.5 * ew
    ecy = ey1 + 0.5 * eh
    gw = gx2 - gx1 + 1.0
    gh = gy2 - gy1 + 1.0
    gcx = gx1 + 0.5 * gw
    gcy = gy1 + 0.5 * gh
    dx = ((gcx - ecx) / ew - 0.0) / 0.1
    dy = ((gcy - ecy) / eh - 0.0) / 0.1
    dw = (jnp.log(gw / ew) - 0.0) / 0.2
    dh = (jnp.log(gh / eh) - 0.0) / 0.2
    tgt = jnp.concatenate([dx, dy, dw, dh], axis=0)
    pos = lab > 0.0
    bt = jnp.where(pos, tgt, 0.0)
    ins = jnp.where(pos, jnp.full(tgt.shape, 1.0, jnp.float32), 0.0)
    outw = jnp.where(ins > 0.0, 1.0, 0.0)
    o_ref[...] = jnp.concatenate([bt, ins, outw], axis=0)


def _run_transform(sc_out):
    return pl.pallas_call(
        _transform_body,
        out_shape=jax.ShapeDtypeStruct((12, ROIS_PER_IMAGE), jnp.float32),
    )(sc_out)


@jax.jit
def kernel(all_rois, gt_boxes, num_boxes, gt_poses):
    del num_boxes
    coords = jnp.concatenate(
        [all_rois[0, :, 1:5], gt_boxes[0, :, :4]], axis=0)
    pad = jnp.full((N_ROWS - coords.shape[0], 4), -1e4, jnp.float32)
    coords = jnp.concatenate([coords, pad], axis=0)
    bx1, by1, bx2, by2 = (coords[:, 0], coords[:, 1],
                          coords[:, 2], coords[:, 3])
    gt_pack = jnp.zeros((8, 32), jnp.float32)
    gt_pack = gt_pack.at[0:4, :N_GT].set(gt_boxes[0, :, :4].T)
    gt_pack = gt_pack.at[4, :N_GT].set(gt_boxes[0, :, 4])
    gt_pack = gt_pack.at[5, :N_GT].set(gt_poses[0])

    mo, bj = _run_iou(bx1, by1, bx2, by2, gt_pack)
    sc_out = _run_sc(mo, bj, bx1, by1, bx2, by2, gt_pack)
    tc_out = _run_transform(sc_out)

    rois_keep = jnp.concatenate(
        [jnp.zeros((ROIS_PER_IMAGE, 1), jnp.float32), sc_out[0:4].T], axis=1)
    labels_keep = sc_out[4]
    poses_keep = sc_out[5]
    bbox_targets = tc_out[0:4].T
    inside = tc_out[4:8].T
    outside = tc_out[8:12].T
    return (rois_keep[None], labels_keep[None], bbox_targets[None],
            inside[None], outside[None], poses_keep[None])
